# Initial kernel scaffold; baseline (speedup 1.0000x reference)
import jax
import jax.numpy as jnp
from jax.experimental import pallas as pl


def kernel(x, embedding_table):
    raise NotImplementedError("write your pallas kernel here")


# speedup vs baseline: 2.4089x; 2.4089x over previous
import functools

import jax
import jax.numpy as jnp
from jax import lax
from jax.experimental import pallas as pl
from jax.experimental.pallas import tpu as pltpu

_EMB_DIM = 64
_NUM_EMB = 1024
_BIG_I32 = 2**30


def _vq_body(x_ref, e_ref, q_ref, idx_ref, loss_ref, ent_ref,
             counts_acc, loss_acc, *, n_steps, n_tokens, n_total):
    b = pl.program_id(0)

    x_t = x_ref[0]
    e = e_ref[...]

    xsq = jnp.sum(x_t * x_t, axis=1, keepdims=True)
    esq = jnp.sum(e * e, axis=0, keepdims=True)
    scores = jnp.dot(x_t, e, preferred_element_type=jnp.float32)
    dist = xsq - 2.0 * scores + esq

    m = jnp.min(dist, axis=1, keepdims=True)
    iota_k = lax.broadcasted_iota(jnp.int32, dist.shape, 1)
    idx = jnp.min(jnp.where(dist == m, iota_k, _BIG_I32),
                  axis=1, keepdims=True)
    idx_ref[0] = idx

    oh = (iota_k == idx).astype(jnp.float32)
    q_ref[0] = lax.dot_general(
        e, oh, (((1,), (1,)), ((), ())),
        preferred_element_type=jnp.float32)

    part = jnp.sum(m)
    cpart = jnp.sum(oh, axis=0, keepdims=True)

    @pl.when(b == 0)
    def _init():
        loss_acc[0, 0] = part
        counts_acc[...] = cpart

    @pl.when(b > 0)
    def _acc():
        loss_acc[0, 0] += part
        counts_acc[...] += cpart

    @pl.when(b == n_steps - 1)
    def _fin():
        loss_ref[0, 0] = loss_acc[0, 0] / n_total
        probs = counts_acc[...] / jnp.float32(n_tokens)
        ent_ref[0, 0] = -jnp.sum(probs * jnp.log(probs + 1e-10))


def kernel(x, embedding_table):
    B, C, H, W = x.shape
    hw = H * W
    n_tokens = B * hw
    x_t = jnp.transpose(x.reshape(B, C, hw), (0, 2, 1))

    body = functools.partial(_vq_body, n_steps=B, n_tokens=n_tokens,
                             n_total=float(n_tokens * C))

    q, idx, loss, ent = pl.pallas_call(
        body,
        grid=(B,),
        in_specs=[
            pl.BlockSpec((1, hw, C), lambda b: (b, 0, 0)),
            pl.BlockSpec((C, _NUM_EMB), lambda b: (0, 0)),
        ],
        out_specs=[
            pl.BlockSpec((1, C, hw), lambda b: (b, 0, 0)),
            pl.BlockSpec((1, hw, 1), lambda b: (b, 0, 0)),
            pl.BlockSpec((1, 1), lambda b: (0, 0),
                         memory_space=pltpu.MemorySpace.SMEM),
            pl.BlockSpec((1, 1), lambda b: (0, 0),
                         memory_space=pltpu.MemorySpace.SMEM),
        ],
        out_shape=[
            jax.ShapeDtypeStruct((B, C, hw), jnp.float32),
            jax.ShapeDtypeStruct((B, hw, 1), jnp.int32),
            jax.ShapeDtypeStruct((1, 1), jnp.float32),
            jax.ShapeDtypeStruct((1, 1), jnp.float32),
        ],
        scratch_shapes=[
            pltpu.VMEM((1, _NUM_EMB), jnp.float32),
            pltpu.SMEM((1, 1), jnp.float32),
        ],
    )(x_t, embedding_table)

    quantized = q.reshape(B, C, H, W)
    loss_s = loss[0, 0]
    return (quantized, loss_s, loss_s, ent[0, 0], idx.reshape(B, hw))


# speedup vs baseline: 2.4924x; 1.0346x over previous
import functools

import jax
import jax.numpy as jnp
from jax import lax
from jax.experimental import pallas as pl
from jax.experimental.pallas import tpu as pltpu

_NUM_EMB = 1024


def _vq_body(x_ref, e2_ref, q_ref, idx_ref, loss_ref, ent_ref,
             counts_acc, loss_acc, iota_scr, *, n_steps, n_tokens, n_total):
    b = pl.program_id(0)

    x_t = x_ref[0]
    e2 = e2_ref[...]

    @pl.when(b == 0)
    def _mk_iota():
        iota_scr[...] = lax.broadcasted_iota(
            jnp.int32, iota_scr.shape, 1).astype(jnp.float32)

    xsq = jnp.sum(x_t * x_t, axis=1, keepdims=True)
    esq = 0.25 * jnp.sum(e2 * e2, axis=0, keepdims=True)
    scores = jnp.dot(x_t, e2, preferred_element_type=jnp.float32)
    dist = (xsq + scores) + esq

    m = jnp.min(dist, axis=1, keepdims=True)
    iota_k = iota_scr[...]
    idx_f = jnp.min(jnp.where(dist == m, iota_k, 2048.0),
                    axis=1, keepdims=True)
    idx_ref[0] = idx_f.astype(jnp.int32)

    oh = (iota_k == idx_f).astype(jnp.float32)
    q_ref[0] = -0.5 * lax.dot_general(
        e2, oh, (((1,), (1,)), ((), ())),
        preferred_element_type=jnp.float32)

    part = jnp.sum(m)
    cpart = jnp.sum(oh, axis=0, keepdims=True)

    @pl.when(b == 0)
    def _init():
        loss_acc[0, 0] = part
        counts_acc[...] = cpart

    @pl.when(b > 0)
    def _acc():
        loss_acc[0, 0] += part
        counts_acc[...] += cpart

    @pl.when(b == n_steps - 1)
    def _fin():
        loss_ref[0, 0] = loss_acc[0, 0] / n_total
        probs = counts_acc[...] / jnp.float32(n_tokens)
        ent_ref[0, 0] = -jnp.sum(probs * jnp.log(probs + 1e-10))


def kernel(x, embedding_table):
    B, C, H, W = x.shape
    hw = H * W
    n_tokens = B * hw
    x_t = jnp.transpose(x.reshape(B, C, hw), (0, 2, 1))
    em2 = -2.0 * embedding_table

    body = functools.partial(_vq_body, n_steps=B, n_tokens=n_tokens,
                             n_total=float(n_tokens * C))

    q, idx, loss, ent = pl.pallas_call(
        body,
        grid=(B,),
        in_specs=[
            pl.BlockSpec((1, hw, C), lambda b: (b, 0, 0)),
            pl.BlockSpec((C, _NUM_EMB), lambda b: (0, 0)),
        ],
        out_specs=[
            pl.BlockSpec((1, C, hw), lambda b: (b, 0, 0)),
            pl.BlockSpec((1, hw, 1), lambda b: (b, 0, 0)),
            pl.BlockSpec((1, 1), lambda b: (0, 0),
                         memory_space=pltpu.MemorySpace.SMEM),
            pl.BlockSpec((1, 1), lambda b: (0, 0),
                         memory_space=pltpu.MemorySpace.SMEM),
        ],
        out_shape=[
            jax.ShapeDtypeStruct((B, C, hw), jnp.float32),
            jax.ShapeDtypeStruct((B, hw, 1), jnp.int32),
            jax.ShapeDtypeStruct((1, 1), jnp.float32),
            jax.ShapeDtypeStruct((1, 1), jnp.float32),
        ],
        scratch_shapes=[
            pltpu.VMEM((1, _NUM_EMB), jnp.float32),
            pltpu.SMEM((1, 1), jnp.float32),
            pltpu.VMEM((1024, _NUM_EMB), jnp.float32),
        ],
    )(x_t, em2)

    quantized = q.reshape(B, C, H, W)
    loss_s = loss[0, 0]
    return (quantized, loss_s, loss_s, ent[0, 0], idx.reshape(B, hw))
